# baseline (speedup 1.0000x reference)
import jax
import jax.numpy as jnp
from jax import lax
from jax.experimental import pallas as pl
from jax.experimental.pallas import tpu as pltpu
from jax.experimental.pallas import tpu_sc as plsc

_N_BOXES = 5000
_NW = 16
_CHUNK = 320
_N_PAD = _NW * _CHUNK
_NV = _CHUNK // 16
_IOU_THRESHOLD = 0.5
_MAX_OUT = 100
_OUT_PAD = 128
_IMG_SIZE = 512.0
_RED = _NW * 16
_BIG = float(_N_PAD)
_EPS25 = float(2.0 ** -25)


def _sc_body(y1h, x1h, y2h, x2h, sh, outh,
             fy1, fx1, fy2, fx2,
             sy1, sx1, sy2, sx2, sarea, sms,
             lv, gred, sred, outv):
    wid = lax.axis_index("s")
    base = wid * _CHUNK

    pltpu.sync_copy(y1h, fy1)
    pltpu.sync_copy(x1h, fx1)
    pltpu.sync_copy(y2h, fy2)
    pltpu.sync_copy(x2h, fx2)
    pltpu.sync_copy(y1h.at[pl.ds(base, _CHUNK)], sy1)
    pltpu.sync_copy(x1h.at[pl.ds(base, _CHUNK)], sx1)
    pltpu.sync_copy(y2h.at[pl.ds(base, _CHUNK)], sy2)
    pltpu.sync_copy(x2h.at[pl.ds(base, _CHUNK)], sx2)
    pltpu.sync_copy(sh.at[pl.ds(base, _CHUNK)], sms)

    lane = lax.iota(jnp.int32, 16)

    def top2_insert(v, gix, v1, i1, v2, i2):
        upd1 = v > v1
        upd2 = v > v2
        v2n = jnp.where(upd1, v1, jnp.where(upd2, v, v2))
        i2n = jnp.where(upd1, i1, jnp.where(upd2, gix, i2))
        v1n = jnp.where(upd1, v, v1)
        i1n = jnp.where(upd1, gix, i1)
        return v1n, i1n, v2n, i2n

    def top2_scalarize(v1, i1, v2, i2):
        i1f = i1.astype(jnp.float32)
        i2f = i2.astype(jnp.float32)
        m1 = jnp.max(v1)
        s1 = jnp.min(jnp.where(v1 == m1, i1f, _BIG))
        c2v = jnp.where(i1f == s1, -1.0, v1)
        m2 = jnp.maximum(jnp.max(c2v), jnp.max(v2))
        s2 = jnp.minimum(jnp.min(jnp.where(c2v == m2, i1f, _BIG)),
                         jnp.min(jnp.where(v2 == m2, i2f, _BIG)))
        return m1, s1, m2, s2

    v1 = jnp.full((16,), -1.0, jnp.float32)
    i1 = jnp.zeros((16,), jnp.int32)
    v2 = jnp.full((16,), -1.0, jnp.float32)
    i2 = jnp.zeros((16,), jnp.int32)
    ms_regs = []
    for i in range(_NV):
        sl = pl.ds(i * 16, 16)
        a = jnp.clip(sy1[sl], 0.0, _IMG_SIZE)
        b = jnp.clip(sx1[sl], 0.0, _IMG_SIZE)
        c = jnp.clip(sy2[sl], 0.0, _IMG_SIZE)
        d = jnp.clip(sx2[sl], 0.0, _IMG_SIZE)
        sy1[sl] = a
        sx1[sl] = b
        sy2[sl] = c
        sx2[sl] = d
        sarea[sl] = (c - a) * (d - b)
        msv = sms[sl]
        ms_regs.append(msv)
        v1, i1, v2, i2 = top2_insert(msv, lane + (base + i * 16),
                                     v1, i1, v2, i2)
    m1, s1, m2, s2 = top2_scalarize(v1, i1, v2, i2)
    ms_regs = tuple(ms_regs)

    zeros = jnp.zeros((16,), jnp.float32)
    for i in range(_OUT_PAD * 5 // 16):
        outv[pl.ds(i * 16, 16)] = zeros

    def get_box(jsel):
        idxv = jnp.full((16,), jsel, jnp.int32)
        ey1 = plsc.load_gather(fy1, [idxv])[0]
        ex1 = plsc.load_gather(fx1, [idxv])[0]
        ey2 = plsc.load_gather(fy2, [idxv])[0]
        ex2 = plsc.load_gather(fx2, [idxv])[0]
        by1 = jnp.clip(ey1, 0.0, _IMG_SIZE)
        bx1 = jnp.clip(ex1, 0.0, _IMG_SIZE)
        by2 = jnp.clip(ey2, 0.0, _IMG_SIZE)
        bx2 = jnp.clip(ex2, 0.0, _IMG_SIZE)
        return by1, bx1, by2, bx2, (by2 - by1) * (bx2 - bx1)

    def cond_fn(carry):
        it, kcnt, done = carry[:3]
        return (kcnt < _MAX_OUT) & (done == 0)

    def body_fn(carry):
        it, kcnt, done, m1, s1, m2, s2 = carry[:7]
        ms_regs = carry[7:]
        lv[...] = jnp.where(lane == 0, m1,
                  jnp.where(lane == 1, s1,
                  jnp.where(lane == 2, m2,
                  jnp.where(lane == 3, s2, 0.0))))
        off = (it % 2) * _RED
        pltpu.sync_copy(lv, sred.at[pl.ds(off + wid * 16, 16)])
        plsc.subcore_barrier()
        pltpu.sync_copy(sred.at[pl.ds(off, _RED)], gred)

        g1v = plsc.load_gather(gred, [lane * 16])
        g1i = plsc.load_gather(gred, [lane * 16 + 1])
        g2v = plsc.load_gather(gred, [lane * 16 + 2])
        g2i = plsc.load_gather(gred, [lane * 16 + 3])
        bv1 = jnp.max(g1v)
        js1f = jnp.min(jnp.where(g1v == bv1, g1i, _BIG))
        c2v = jnp.where(g1i == js1f, -1.0, g1v)
        bv2 = jnp.maximum(jnp.max(c2v), jnp.max(g2v))
        js2f = jnp.minimum(jnp.min(jnp.where(c2v == bv2, g1i, _BIG)),
                           jnp.min(jnp.where(g2v == bv2, g2i, _BIG)))

        def do():
            jsel1 = js1f.astype(jnp.int32)
            jsel2 = jnp.minimum(js2f,
                                jnp.float32(_N_PAD - 1)).astype(jnp.int32)
            ay1, ax1, ay2, ax2, aarea = get_box(jsel1)
            by1, bx1, by2, bx2, barea = get_box(jsel2)
            wiy1 = jnp.maximum(ay1, by1)
            wix1 = jnp.maximum(ax1, bx1)
            wiy2 = jnp.minimum(ay2, by2)
            wix2 = jnp.minimum(ax2, bx2)
            winter = (jnp.maximum(wiy2 - wiy1, 0.0)
                      * jnp.maximum(wix2 - wix1, 0.0))
            wu = jnp.maximum(aarea + barea - winter, 1e-8)
            wover = (winter - 0.5 * wu) > wu * _EPS25
            both = ((bv2 > -0.5) & (kcnt < _MAX_OUT - 1)
                    & jnp.logical_not(wover))

            oval1 = (jnp.where(lane == 0, ay1, 0.0)
                     + jnp.where(lane == 1, ax1, 0.0)
                     + jnp.where(lane == 2, ay2, 0.0)
                     + jnp.where(lane == 3, ax2, 0.0)
                     + jnp.where(lane == 4, bv1, 0.0))
            plsc.store_scatter(outv, [kcnt + lane * _OUT_PAD], oval1,
                               mask=lane < 5)
            oval2 = (jnp.where(lane == 0, by1, 0.0)
                     + jnp.where(lane == 1, bx1, 0.0)
                     + jnp.where(lane == 2, by2, 0.0)
                     + jnp.where(lane == 3, bx2, 0.0)
                     + jnp.where(lane == 4, bv2, 0.0))
            plsc.store_scatter(outv, [kcnt + 1 + lane * _OUT_PAD], oval2,
                               mask=(lane < 5) & both)

            v1 = jnp.full((16,), -1.0, jnp.float32)
            i1 = jnp.zeros((16,), jnp.int32)
            v2 = jnp.full((16,), -1.0, jnp.float32)
            i2 = jnp.zeros((16,), jnp.int32)
            ms_new = []
            for i in range(_NV):
                sl = pl.ds(i * 16, 16)
                ty1 = sy1[sl]
                tx1 = sx1[sl]
                ty2 = sy2[sl]
                tx2 = sx2[sl]
                tarea = sarea[sl]
                iy1 = jnp.maximum(ty1, ay1)
                ix1 = jnp.maximum(tx1, ax1)
                iy2 = jnp.minimum(ty2, ay2)
                ix2 = jnp.minimum(tx2, ax2)
                inter1 = (jnp.maximum(iy2 - iy1, 0.0)
                          * jnp.maximum(ix2 - ix1, 0.0))
                u1 = jnp.maximum(tarea + aarea - inter1, 1e-8)
                ov1 = (inter1 - 0.5 * u1) > u1 * _EPS25
                jy1 = jnp.maximum(ty1, by1)
                jx1 = jnp.maximum(tx1, bx1)
                jy2 = jnp.minimum(ty2, by2)
                jx2 = jnp.minimum(tx2, bx2)
                inter2 = (jnp.maximum(jy2 - jy1, 0.0)
                          * jnp.maximum(jx2 - jx1, 0.0))
                u2 = jnp.maximum(tarea + barea - inter2, 1e-8)
                ov2 = (inter2 - 0.5 * u2) > u2 * _EPS25
                gix = lane + (base + i * 16)
                sup = (ov1 | (gix == jsel1)
                       | (both & (ov2 | (gix == jsel2))))
                msn = jnp.where(sup, -1.0, ms_regs[i])
                ms_new.append(msn)
                v1, i1, v2, i2 = top2_insert(msn, gix, v1, i1, v2, i2)
            m1n, s1n, m2n, s2n = top2_scalarize(v1, i1, v2, i2)
            kn = kcnt + 1 + jnp.where(both, 1, 0)
            return (kn, jnp.bool_(False), m1n, s1n, m2n, s2n) + tuple(ms_new)

        def skip():
            return (kcnt, jnp.bool_(True), m1, s1, m2, s2) + tuple(ms_regs)

        res = lax.cond(bv1 > -0.5, do, skip)
        kn, done_n, m1n, s1n, m2n, s2n = res[:6]
        return (it + 1, kn, done_n, m1n, s1n, m2n, s2n) + tuple(res[6:])

    lax.while_loop(cond_fn, body_fn,
                   (jnp.int32(0), jnp.int32(0), jnp.bool_(False),
                    m1, s1, m2, s2) + ms_regs)

    @pl.when(wid == 0)
    def _():
        pltpu.sync_copy(outv, outh)


def _make_sc_call(interpret=False):
    mesh = plsc.VectorSubcoreMesh(core_axis_name="c", subcore_axis_name="s",
                                  num_cores=1, num_subcores=_NW)
    return pl.kernel(
        _sc_body,
        out_type=jax.ShapeDtypeStruct((_OUT_PAD * 5,), jnp.float32),
        mesh=mesh,
        scratch_types=[
            pltpu.VMEM((_N_PAD,), jnp.float32),
            pltpu.VMEM((_N_PAD,), jnp.float32),
            pltpu.VMEM((_N_PAD,), jnp.float32),
            pltpu.VMEM((_N_PAD,), jnp.float32),
            pltpu.VMEM((_CHUNK,), jnp.float32),
            pltpu.VMEM((_CHUNK,), jnp.float32),
            pltpu.VMEM((_CHUNK,), jnp.float32),
            pltpu.VMEM((_CHUNK,), jnp.float32),
            pltpu.VMEM((_CHUNK,), jnp.float32),
            pltpu.VMEM((_CHUNK,), jnp.float32),
            pltpu.VMEM((16,), jnp.float32),
            pltpu.VMEM((_RED,), jnp.float32),
            pltpu.VMEM_SHARED((2 * _RED,), jnp.float32),
            pltpu.VMEM((_OUT_PAD * 5,), jnp.float32),
        ],
        compiler_params=pltpu.CompilerParams(needs_layout_passes=False),
        interpret=interpret,
    )


def kernel(boxes, scores):
    pad = _N_PAD - _N_BOXES
    y1 = jnp.pad(boxes[:, 0], (0, pad))
    x1 = jnp.pad(boxes[:, 1], (0, pad))
    y2 = jnp.pad(boxes[:, 2], (0, pad))
    x2 = jnp.pad(boxes[:, 3], (0, pad))
    s = jnp.pad(scores, (0, pad), constant_values=-1.0)
    outv = _make_sc_call()(y1, x1, y2, x2, s)
    return outv.reshape(5, _OUT_PAD).T[:_MAX_OUT]


# speedup vs baseline: 1.1187x; 1.1187x over previous
import jax
import jax.numpy as jnp
from jax import lax
from jax.experimental import pallas as pl
from jax.experimental.pallas import tpu as pltpu
from jax.experimental.pallas import tpu_sc as plsc

_N_BOXES = 5000
_NW = 16
_CHUNK = 320
_N_PAD = _NW * _CHUNK
_NV = _CHUNK // 16
_IOU_THRESHOLD = 0.5
_MAX_OUT = 100
_OUT_PAD = 128
_IMG_SIZE = 512.0
_RED = _NW * 16
_BIG = float(_N_PAD)
_EPS25 = float(2.0 ** -25)


def _over_v(inter, union):
    u = jnp.maximum(union, 1e-8)
    return (inter - 0.5 * u) > u * _EPS25


def _top4_insert(x, gix, v, ix):
    c0 = x > v[0]
    c1 = x > v[1]
    c2 = x > v[2]
    c3 = x > v[3]
    v3n = jnp.where(c2, v[2], jnp.where(c3, x, v[3]))
    i3n = jnp.where(c2, ix[2], jnp.where(c3, gix, ix[3]))
    v2n = jnp.where(c1, v[1], jnp.where(c2, x, v[2]))
    i2n = jnp.where(c1, ix[1], jnp.where(c2, gix, ix[2]))
    v1n = jnp.where(c0, v[0], jnp.where(c1, x, v[1]))
    i1n = jnp.where(c0, ix[0], jnp.where(c1, gix, ix[1]))
    v0n = jnp.where(c0, x, v[0])
    i0n = jnp.where(c0, gix, ix[0])
    return [v0n, v1n, v2n, v3n], [i0n, i1n, i2n, i3n]


def _extract4(v, ixf):
    depth = jnp.zeros((16,), jnp.int32)
    res = []
    for _ in range(4):
        hv = jnp.where(depth == 0, v[0],
             jnp.where(depth == 1, v[1],
             jnp.where(depth == 2, v[2],
             jnp.where(depth == 3, v[3], -1.0))))
        hi = jnp.where(depth == 1, ixf[1],
             jnp.where(depth == 2, ixf[2],
             jnp.where(depth == 3, ixf[3], ixf[0])))
        m = jnp.max(hv)
        sidx = jnp.min(jnp.where(hv == m, hi, _BIG))
        res.append((m, sidx))
        depth = depth + ((hv == m) & (hi == sidx)).astype(jnp.int32)
    return res


def _sc_body(y1h, x1h, y2h, x2h, sh, outh,
             fy1, fx1, fy2, fx2,
             sy1, sx1, sy2, sx2, sarea, sms,
             lv, gred, sred, outv):
    wid = lax.axis_index("s")
    base = wid * _CHUNK

    pltpu.sync_copy(y1h, fy1)
    pltpu.sync_copy(x1h, fx1)
    pltpu.sync_copy(y2h, fy2)
    pltpu.sync_copy(x2h, fx2)
    pltpu.sync_copy(y1h.at[pl.ds(base, _CHUNK)], sy1)
    pltpu.sync_copy(x1h.at[pl.ds(base, _CHUNK)], sx1)
    pltpu.sync_copy(y2h.at[pl.ds(base, _CHUNK)], sy2)
    pltpu.sync_copy(x2h.at[pl.ds(base, _CHUNK)], sx2)
    pltpu.sync_copy(sh.at[pl.ds(base, _CHUNK)], sms)

    lane = lax.iota(jnp.int32, 16)

    v = [jnp.full((16,), -1.0, jnp.float32) for _ in range(4)]
    ix = [jnp.zeros((16,), jnp.int32) for _ in range(4)]
    for i in range(_NV):
        sl = pl.ds(i * 16, 16)
        a = jnp.clip(sy1[sl], 0.0, _IMG_SIZE)
        b = jnp.clip(sx1[sl], 0.0, _IMG_SIZE)
        c = jnp.clip(sy2[sl], 0.0, _IMG_SIZE)
        d = jnp.clip(sx2[sl], 0.0, _IMG_SIZE)
        sy1[sl] = a
        sx1[sl] = b
        sy2[sl] = c
        sx2[sl] = d
        sarea[sl] = (c - a) * (d - b)
        v, ix = _top4_insert(sms[sl], lane + (base + i * 16), v, ix)
    loc0 = _extract4(v, [t.astype(jnp.float32) for t in ix])

    zeros = jnp.zeros((16,), jnp.float32)
    for i in range(_OUT_PAD * 5 // 16):
        outv[pl.ds(i * 16, 16)] = zeros

    def get_box(jself):
        jsel = jnp.minimum(jself, _BIG - 1.0).astype(jnp.int32)
        idxv = jnp.full((16,), jsel, jnp.int32)
        ey1 = plsc.load_gather(fy1, [idxv])[0]
        ex1 = plsc.load_gather(fx1, [idxv])[0]
        ey2 = plsc.load_gather(fy2, [idxv])[0]
        ex2 = plsc.load_gather(fx2, [idxv])[0]
        by1 = jnp.clip(ey1, 0.0, _IMG_SIZE)
        bx1 = jnp.clip(ex1, 0.0, _IMG_SIZE)
        by2 = jnp.clip(ey2, 0.0, _IMG_SIZE)
        bx2 = jnp.clip(ex2, 0.0, _IMG_SIZE)
        return jsel, by1, bx1, by2, bx2, (by2 - by1) * (bx2 - bx1)

    def cond_fn(carry):
        kcnt, done = carry[1], carry[2]
        return (kcnt < _MAX_OUT) & (done == 0)

    def body_fn(carry):
        it, kcnt, done = carry[:3]
        ms8 = carry[3:]
        pub = ms8[0]
        for k in range(1, 8):
            pub = jnp.where(lane == k, ms8[k], pub)
        lv[...] = jnp.where(lane < 8, pub, 0.0)
        off = (it % 2) * _RED
        pltpu.sync_copy(lv, sred.at[pl.ds(off + wid * 16, 16)])
        plsc.subcore_barrier()
        pltpu.sync_copy(sred.at[pl.ds(off, _RED)], gred)

        gv = [plsc.load_gather(gred, [lane * 16 + 2 * r]) for r in range(4)]
        gi = [plsc.load_gather(gred, [lane * 16 + 2 * r + 1]) for r in range(4)]
        win = _extract4(gv, gi)
        bv1 = win[0][0]

        def do():
            boxes = [get_box(w[1]) for w in win]

            def pair_over(a, b):
                iy1 = jnp.maximum(a[1], b[1])
                ix1 = jnp.maximum(a[2], b[2])
                iy2 = jnp.minimum(a[3], b[3])
                ix2 = jnp.minimum(a[4], b[4])
                it_ = (jnp.maximum(iy2 - iy1, 0.0)
                       * jnp.maximum(ix2 - ix1, 0.0))
                return _over_v(it_, a[5] + b[5] - it_)

            P = {}
            for a in range(4):
                for b in range(a + 1, 4):
                    P[(a, b)] = pair_over(boxes[a], boxes[b])

            e1 = jnp.bool_(True)
            e2 = ((win[1][0] > -0.5) & (kcnt + 1 <= _MAX_OUT - 1)
                  & jnp.logical_not(P[(0, 1)]))
            pos2 = kcnt + 1 + e2.astype(jnp.int32)
            e3 = ((win[2][0] > -0.5) & (pos2 <= _MAX_OUT - 1)
                  & jnp.logical_not(P[(0, 2)])
                  & jnp.logical_not(e2 & P[(1, 2)]))
            pos3 = pos2 + e3.astype(jnp.int32)
            e4 = ((win[3][0] > -0.5) & (pos3 <= _MAX_OUT - 1)
                  & jnp.logical_not(P[(0, 3)])
                  & jnp.logical_not(e2 & P[(1, 3)])
                  & jnp.logical_not(e3 & P[(2, 3)]))
            emits = (e1, e2, e3, e4)
            slots = (kcnt, kcnt + 1, pos2, pos3)

            for k in range(4):
                bx = boxes[k]
                ov = (jnp.where(lane == 0, bx[1], 0.0)
                      + jnp.where(lane == 1, bx[2], 0.0)
                      + jnp.where(lane == 2, bx[3], 0.0)
                      + jnp.where(lane == 3, bx[4], 0.0)
                      + jnp.where(lane == 4, win[k][0], 0.0))
                plsc.store_scatter(outv, [slots[k] + lane * _OUT_PAD], ov,
                                   mask=(lane < 5) & emits[k])

            v = [jnp.full((16,), -1.0, jnp.float32) for _ in range(4)]
            ix = [jnp.zeros((16,), jnp.int32) for _ in range(4)]
            for i in range(_NV):
                sl = pl.ds(i * 16, 16)
                ty1 = sy1[sl]
                tx1 = sx1[sl]
                ty2 = sy2[sl]
                tx2 = sx2[sl]
                tarea = sarea[sl]
                gix = lane + (base + i * 16)
                sup = jnp.zeros((16,), jnp.bool_)
                for k in range(4):
                    bx = boxes[k]
                    iy1 = jnp.maximum(ty1, bx[1])
                    ixx1 = jnp.maximum(tx1, bx[2])
                    iy2 = jnp.minimum(ty2, bx[3])
                    ixx2 = jnp.minimum(tx2, bx[4])
                    it_ = (jnp.maximum(iy2 - iy1, 0.0)
                           * jnp.maximum(ixx2 - ixx1, 0.0))
                    ovk = _over_v(it_, tarea + bx[5] - it_)
                    sup = sup | (emits[k] & (ovk | (gix == bx[0])))
                msn = jnp.where(sup, -1.0, sms[sl])
                sms[sl] = msn
                v, ix = _top4_insert(msn, gix, v, ix)
            nloc = _extract4(v, [t.astype(jnp.float32) for t in ix])
            kn = pos3 + e4.astype(jnp.int32)
            flat = []
            for m, si in nloc:
                flat += [m, si]
            return (kn, jnp.int32(0)) + tuple(flat)

        def skip():
            return (kcnt, jnp.int32(1)) + tuple(ms8)

        res = lax.cond(bv1 > -0.5, do, skip)
        return (it + 1,) + tuple(res)

    flat0 = []
    for m, si in loc0:
        flat0 += [m, si]
    lax.while_loop(cond_fn, body_fn,
                   (jnp.int32(0), jnp.int32(0), jnp.int32(0)) + tuple(flat0))

    @pl.when(wid == 0)
    def _():
        pltpu.sync_copy(outv, outh)


def _make_sc_call(interpret=False):
    mesh = plsc.VectorSubcoreMesh(core_axis_name="c", subcore_axis_name="s",
                                  num_cores=1, num_subcores=_NW)
    return pl.kernel(
        _sc_body,
        out_type=jax.ShapeDtypeStruct((_OUT_PAD * 5,), jnp.float32),
        mesh=mesh,
        scratch_types=[
            pltpu.VMEM((_N_PAD,), jnp.float32),
            pltpu.VMEM((_N_PAD,), jnp.float32),
            pltpu.VMEM((_N_PAD,), jnp.float32),
            pltpu.VMEM((_N_PAD,), jnp.float32),
            pltpu.VMEM((_CHUNK,), jnp.float32),
            pltpu.VMEM((_CHUNK,), jnp.float32),
            pltpu.VMEM((_CHUNK,), jnp.float32),
            pltpu.VMEM((_CHUNK,), jnp.float32),
            pltpu.VMEM((_CHUNK,), jnp.float32),
            pltpu.VMEM((_CHUNK,), jnp.float32),
            pltpu.VMEM((16,), jnp.float32),
            pltpu.VMEM((_RED,), jnp.float32),
            pltpu.VMEM_SHARED((2 * _RED,), jnp.float32),
            pltpu.VMEM((_OUT_PAD * 5,), jnp.float32),
        ],
        compiler_params=pltpu.CompilerParams(needs_layout_passes=False),
        interpret=interpret,
    )


def kernel(boxes, scores):
    pad = _N_PAD - _N_BOXES
    y1 = jnp.pad(boxes[:, 0], (0, pad))
    x1 = jnp.pad(boxes[:, 1], (0, pad))
    y2 = jnp.pad(boxes[:, 2], (0, pad))
    x2 = jnp.pad(boxes[:, 3], (0, pad))
    s = jnp.pad(scores, (0, pad), constant_values=-1.0)
    outv = _make_sc_call()(y1, x1, y2, x2, s)
    return outv.reshape(5, _OUT_PAD).T[:_MAX_OUT]
